# baseline (speedup 1.0000x reference)
import jax
import jax.numpy as jnp
from jax import lax
from jax.experimental import pallas as pl
from jax.experimental.pallas import tpu as pltpu

_SRC_X = 2048
_SRC_Y = 2048
_ROWS = 4096
_COLS = 4096

_C = 1024
_NCH = _ROWS // _C
_D = 3
_ISRC = _SRC_X // _C
_LR = _SRC_X % _C
_LR8 = (_LR // 8) * 8


def _body(bt_ref, b_any, o_any, *rest):
    bufs = rest[:_D]
    in_sems = rest[_D:2 * _D]
    out_sems = rest[2 * _D:]

    def in_copy(i, d):
        return pltpu.make_async_copy(
            b_any.at[pl.ds(i * _C, _C), :], bufs[d], in_sems[d])

    def out_copy(i, d):
        return pltpu.make_async_copy(
            bufs[d], o_any.at[pl.ds(i * _C, _C), :], out_sems[d])

    for i in range(_D):
        in_copy(i, i).start()

    for i in range(_NCH):
        d = i % _D
        in_copy(i, d).wait()
        if i == _ISRC:
            ri = lax.broadcasted_iota(jnp.int32, (8, 128), 0)
            ci = lax.broadcasted_iota(jnp.int32, (8, 128), 1)
            sub = bufs[d][pl.ds(_LR8, 8), pl.ds(_SRC_Y, 128)]
            bufs[d][pl.ds(_LR8, 8), pl.ds(_SRC_Y, 128)] = jnp.where(
                (ri == _LR - _LR8) & (ci == 0), bt_ref[0, 0], sub)
        out_copy(i, d).start()
        nxt = i + _D
        if nxt < _NCH:
            out_copy(i, d).wait()
            in_copy(nxt, d).start()

    for i in range(_NCH - _D, _NCH):
        out_copy(i, i % _D).wait()


@jax.jit
def _scatter_copy(bt, b2d):
    return pl.pallas_call(
        _body,
        in_specs=[
            pl.BlockSpec(memory_space=pltpu.SMEM),
            pl.BlockSpec(memory_space=pl.ANY),
        ],
        out_specs=pl.BlockSpec(memory_space=pl.ANY),
        out_shape=jax.ShapeDtypeStruct((_ROWS, _COLS), jnp.float32),
        scratch_shapes=(
            [pltpu.VMEM((_C, _COLS), jnp.float32) for _ in range(_D)]
            + [pltpu.SemaphoreType.DMA for _ in range(2 * _D)]
        ),
    )(bt, b2d)


def kernel(B, Bt):
    out = _scatter_copy(Bt, B.reshape(_ROWS, _COLS))
    return out.reshape(B.shape)


# speedup vs baseline: 1.0020x; 1.0020x over previous
import jax
import jax.numpy as jnp
from jax import lax
from jax.experimental import pallas as pl
from jax.experimental.pallas import tpu as pltpu

_SRC_X = 2048
_SRC_Y = 2048
_ROWS = 4096
_COLS = 4096

_BUF_ROWS = (128, 256, 512, 1024, 1024, 768)
_CHUNKS = (
    (0, 128, 0, None),
    (128, 256, 1, None),
    (384, 512, 2, None),
    (896, 1024, 3, None),
    (1920, 1024, 4, None),
    (2944, 128, 0, 0),
    (3072, 768, 5, None),
    (3840, 256, 1, 1),
)
_ISRC = 4
_LR = _SRC_X - _CHUNKS[_ISRC][0]
_LR8 = (_LR // 8) * 8


def _body(bt_ref, b_any, o_any, *rest):
    nb = len(_BUF_ROWS)
    bufs = rest[:nb]
    in_sems = rest[nb:2 * nb]
    out_sems = rest[2 * nb:]

    def in_copy(i):
        st, rows, b, _ = _CHUNKS[i]
        return pltpu.make_async_copy(
            b_any.at[pl.ds(st, rows), :], bufs[b], in_sems[b])

    def out_copy(i):
        st, rows, b, _ = _CHUNKS[i]
        return pltpu.make_async_copy(
            bufs[b], o_any.at[pl.ds(st, rows), :], out_sems[b])

    for i, (_, _, _, dep) in enumerate(_CHUNKS):
        if dep is None:
            in_copy(i).start()

    started = set(i for i, c in enumerate(_CHUNKS) if c[3] is None)
    for i, (st, rows, b, _) in enumerate(_CHUNKS):
        in_copy(i).wait()
        if i == _ISRC:
            ri = lax.broadcasted_iota(jnp.int32, (8, 128), 0)
            ci = lax.broadcasted_iota(jnp.int32, (8, 128), 1)
            sub = bufs[b][pl.ds(_LR8, 8), pl.ds(_SRC_Y, 128)]
            bufs[b][pl.ds(_LR8, 8), pl.ds(_SRC_Y, 128)] = jnp.where(
                (ri == _LR - _LR8) & (ci == 0), bt_ref[0, 0], sub)
        out_copy(i).start()
        for j, (_, _, _, dep) in enumerate(_CHUNKS):
            if dep is not None and j not in started and dep < i:
                out_copy(dep).wait()
                in_copy(j).start()
                started.add(j)

    deps = set(c[3] for c in _CHUNKS if c[3] is not None)
    for i in range(len(_CHUNKS)):
        if i not in deps:
            out_copy(i).wait()


@jax.jit
def _scatter_copy(bt, b2d):
    nb = len(_BUF_ROWS)
    return pl.pallas_call(
        _body,
        in_specs=[
            pl.BlockSpec(memory_space=pltpu.SMEM),
            pl.BlockSpec(memory_space=pl.ANY),
        ],
        out_specs=pl.BlockSpec(memory_space=pl.ANY),
        out_shape=jax.ShapeDtypeStruct((_ROWS, _COLS), jnp.float32),
        scratch_shapes=(
            [pltpu.VMEM((r, _COLS), jnp.float32) for r in _BUF_ROWS]
            + [pltpu.SemaphoreType.DMA for _ in range(2 * nb)]
        ),
    )(bt, b2d)


def kernel(B, Bt):
    out = _scatter_copy(Bt, B.reshape(_ROWS, _COLS))
    return out.reshape(B.shape)
